# baseline (speedup 1.0000x reference)
import jax
import jax.numpy as jnp
from jax.experimental import pallas as pl
from jax.experimental.pallas import tpu as pltpu


_C_IN = 96
_E = 32
_C_OUT = _C_IN + _E
_HW = 224 * 224
_NJ = 8
_KC = _HW // _NJ
_NC = 4 * _NJ
_NBUF = 8
_D = 3


def _body(idx_ref, x_ref, w_ref, out_ref, bufs, fill, in_sems, out_sems,
          fill_sem):
    def in_copy(i):
        b, j = divmod(i, _NJ)
        slot = i % _NBUF
        return pltpu.make_async_copy(
            x_ref.at[b, :, pl.ds(j * _KC, _KC)], bufs.at[slot],
            in_sems.at[slot])

    def out_copy(i):
        b, j = divmod(i, _NJ)
        slot = i % _NBUF
        return pltpu.make_async_copy(
            bufs.at[slot],
            out_ref.at[b, pl.ds(0, _C_IN), pl.ds(j * _KC, _KC)],
            out_sems.at[slot])

    def fill_copy(b):
        return pltpu.make_async_copy(
            fill, out_ref.at[b, pl.ds(_C_IN, _E), :], fill_sem)

    for i in range(_D):
        in_copy(i).start()
    w = w_ref[idx_ref[0], :]
    fill[...] = jnp.broadcast_to(w[:, None], (_E, _HW))
    for b in range(4):
        fill_copy(b).start()
    for i in range(_D, _NC + _D):
        if i < _NC:
            if i >= _NBUF:
                out_copy(i - _NBUF).wait()
            in_copy(i).start()
        j = i - _D
        in_copy(j).wait()
        out_copy(j).start()
    for i in range(_NC - _NBUF, _NC):
        out_copy(i).wait()
    for b in range(4):
        fill_copy(b).wait()


def kernel(x, W, view_idx):
    B, C, H, Wd = x.shape
    hw = H * Wd
    x3 = x.reshape(B, C, hw)
    idx = jnp.asarray(view_idx, jnp.int32).reshape(1)

    out3 = pl.pallas_call(
        _body,
        in_specs=[
            pl.BlockSpec(memory_space=pltpu.SMEM),
            pl.BlockSpec(memory_space=pl.ANY),
            pl.BlockSpec(memory_space=pltpu.VMEM),
        ],
        out_specs=pl.BlockSpec(memory_space=pl.ANY),
        out_shape=jax.ShapeDtypeStruct((B, _C_OUT, hw), x.dtype),
        scratch_shapes=[
            pltpu.VMEM((_NBUF, _C_IN, _KC), jnp.float32),
            pltpu.VMEM((_E, _HW), jnp.float32),
            pltpu.SemaphoreType.DMA((_NBUF,)),
            pltpu.SemaphoreType.DMA((_NBUF,)),
            pltpu.SemaphoreType.DMA,
        ],
    )(idx, x3, W)
    return out3.reshape(B, _C_OUT, H, Wd)
